# baseline (speedup 1.0000x reference)
import math

import jax
import jax.numpy as jnp
from jax import lax
from jax.experimental import pallas as pl
from jax.experimental.pallas import tpu as pltpu

_C0 = math.sqrt(2.0 / math.pi)
_C1 = 0.044715

_CH = 256
_NB = 8


def _tc_ring_gelu(xf):
    R, D = xf.shape
    nch = R // _CH

    def body(x_hbm, o_hbm, bin_, bout, isem, osem):
        def in_copy(k, slot):
            return pltpu.make_async_copy(
                x_hbm.at[pl.ds(k * _CH, _CH), :], bin_.at[slot], isem.at[slot])

        def out_copy(k, slot):
            return pltpu.make_async_copy(
                bout.at[slot], o_hbm.at[pl.ds(k * _CH, _CH), :], osem.at[slot])

        for s in range(_NB):
            in_copy(s, s).start()

        def super_step(j, carry):
            for s in range(_NB):
                k = j * _NB + s
                in_copy(k, s).wait()

                @pl.when(k >= _NB)
                def _():
                    out_copy(k - _NB, s).wait()

                x = bin_[s]
                hx = 0.5 * x
                u = x * (_C0 + (_C0 * _C1) * (x * x))
                bout[s] = hx + hx * jnp.tanh(u)
                out_copy(k, s).start()

                @pl.when(k + _NB < nch)
                def _():
                    in_copy(k + _NB, s).start()
            return carry

        lax.fori_loop(0, nch // _NB, super_step, 0)
        for s in range(_NB):
            out_copy(0, s).wait()

    return pl.pallas_call(
        body,
        out_shape=jax.ShapeDtypeStruct((R, D), xf.dtype),
        in_specs=[pl.BlockSpec(memory_space=pltpu.HBM)],
        out_specs=pl.BlockSpec(memory_space=pltpu.HBM),
        scratch_shapes=[
            pltpu.VMEM((_NB, _CH, D), jnp.float32),
            pltpu.VMEM((_NB, _CH, D), jnp.float32),
            pltpu.SemaphoreType.DMA((_NB,)),
            pltpu.SemaphoreType.DMA((_NB,)),
        ],
    )(xf)


def kernel(x, log_k_inject):
    B, T, D = x.shape
    y = _tc_ring_gelu(x.reshape(B * T, D))
    return y.reshape(B, T, D)


# speedup vs baseline: 1.0005x; 1.0005x over previous
import math

import jax
import jax.numpy as jnp
from jax import lax
from jax.experimental import pallas as pl
from jax.experimental.pallas import tpu as pltpu

_C0 = math.sqrt(2.0 / math.pi)
_C1 = 0.044715

_CH = 128
_NB = 8


def _tc_ring_gelu(xf):
    R, D = xf.shape
    nch = R // _CH

    def body(x_hbm, o_hbm, bin_, bout, isem, osem):
        def in_copy(k, slot):
            return pltpu.make_async_copy(
                x_hbm.at[pl.ds(k * _CH, _CH), :], bin_.at[slot], isem.at[slot])

        def out_copy(k, slot):
            return pltpu.make_async_copy(
                bout.at[slot], o_hbm.at[pl.ds(k * _CH, _CH), :], osem.at[slot])

        for s in range(_NB):
            in_copy(s, s).start()

        def super_step(j, carry):
            for s in range(_NB):
                k = j * _NB + s
                in_copy(k, s).wait()

                @pl.when(k >= _NB)
                def _():
                    out_copy(k - _NB, s).wait()

                x = bin_[s]
                hx = 0.5 * x
                u = x * (_C0 + (_C0 * _C1) * (x * x))
                bout[s] = hx + hx * jnp.tanh(u)
                out_copy(k, s).start()

                @pl.when(k + _NB < nch)
                def _():
                    in_copy(k + _NB, s).start()
            return carry

        lax.fori_loop(0, nch // _NB, super_step, 0)
        for s in range(_NB):
            out_copy(0, s).wait()

    return pl.pallas_call(
        body,
        out_shape=jax.ShapeDtypeStruct((R, D), xf.dtype),
        in_specs=[pl.BlockSpec(memory_space=pltpu.HBM)],
        out_specs=pl.BlockSpec(memory_space=pltpu.HBM),
        scratch_shapes=[
            pltpu.VMEM((_NB, _CH, D), jnp.float32),
            pltpu.VMEM((_NB, _CH, D), jnp.float32),
            pltpu.SemaphoreType.DMA((_NB,)),
            pltpu.SemaphoreType.DMA((_NB,)),
        ],
    )(xf)


def kernel(x, log_k_inject):
    B, T, D = x.shape
    y = _tc_ring_gelu(x.reshape(B * T, D))
    return y.reshape(B, T, D)
